# baseline (speedup 1.0000x reference)
import functools

import jax
import jax.numpy as jnp
from jax import lax
from jax.experimental import pallas as pl
from jax.experimental.pallas import tpu as pltpu
from jax.experimental.pallas import tpu_sc as plsc

N = 10000
E = 320000
D = 128

NW = 32
EPW = E // NW
CH = 80
NCH = EPW // CH
NPAD = 10240
SEG = NPAD // 16

ROWB = 1000

_mesh = plsc.VectorSubcoreMesh(core_axis_name="c", subcore_axis_name="s")
_sc_params = pltpu.CompilerParams(needs_layout_passes=False,
                                  use_tc_tiling_on_sc=False)


def _dense_in_body(x_ref, ws_ref, wd_ref, asv_ref, adv_ref,
                   h_ref, as_ref, ad_ref, m_ref, acc):
    i = pl.program_id(0)
    h = jnp.dot(x_ref[...], ws_ref[...], preferred_element_type=jnp.float32)
    h_ref[0] = h[:, :64]
    h_ref[1] = h[:, 64:]
    a_s = jnp.dot(h, asv_ref[...], preferred_element_type=jnp.float32)
    as_ref[...] = a_s
    v_d = jnp.dot(wd_ref[...], adv_ref[...], preferred_element_type=jnp.float32)
    a_d = jnp.dot(x_ref[...], v_d, preferred_element_type=jnp.float32)
    ad_ref[...] = a_d
    bs = jnp.max(a_s)
    bd = jnp.max(a_d)

    @pl.when(i == 0)
    def _():
        acc[0] = bs
        acc[1] = bd

    @pl.when(i > 0)
    def _():
        acc[0] = jnp.maximum(acc[0], bs)
        acc[1] = jnp.maximum(acc[1], bd)

    @pl.when(i == pl.num_programs(0) - 1)
    def _():
        m = acc[0] + acc[1]
        m = jnp.where(m >= 0.0, m, m * 0.2)
        m_ref[...] = jnp.full((8, 128), m, jnp.float32)


_dense_in = pl.pallas_call(
    _dense_in_body,
    grid=(N // ROWB,),
    in_specs=[
        pl.BlockSpec((ROWB, D), lambda i: (i, 0)),
        pl.BlockSpec((D, D), lambda i: (0, 0)),
        pl.BlockSpec((D, D), lambda i: (0, 0)),
        pl.BlockSpec((D, 1), lambda i: (0, 0)),
        pl.BlockSpec((D, 1), lambda i: (0, 0)),
    ],
    out_specs=[
        pl.BlockSpec((2, ROWB, D // 2), lambda i: (0, i, 0)),
        pl.BlockSpec((ROWB, 1), lambda i: (i, 0)),
        pl.BlockSpec((ROWB, 1), lambda i: (i, 0)),
        pl.BlockSpec((8, 128), lambda i: (0, 0)),
    ],
    out_shape=[
        jax.ShapeDtypeStruct((2, N, D // 2), jnp.float32),
        jax.ShapeDtypeStruct((N, 1), jnp.float32),
        jax.ShapeDtypeStruct((N, 1), jnp.float32),
        jax.ShapeDtypeStruct((8, 128), jnp.float32),
    ],
    scratch_shapes=[pltpu.SMEM((2,), jnp.float32)],
)


@functools.partial(
    pl.kernel,
    mesh=_mesh,
    out_type=[
        jax.ShapeDtypeStruct((NW, NCH, CH), jnp.float32),
        jax.ShapeDtypeStruct((2, NPAD), jnp.float32),
    ],
    scratch_types=[
        pltpu.VMEM((N,), jnp.float32),
        pltpu.VMEM((N,), jnp.float32),
        pltpu.VMEM((NCH, CH), jnp.int32),
        pltpu.VMEM((NCH, CH), jnp.int32),
        pltpu.VMEM((NCH, CH), jnp.float32),
        pltpu.VMEM((16,), jnp.float32),
        pltpu.VMEM((SEG,), jnp.float32),
        pltpu.VMEM_SHARED((NPAD,), jnp.float32),
    ],
    compiler_params=_sc_params,
)
def _edge_pass_a(a_src_hbm, a_dst_hbm, edge_hbm, m_hbm,
                 ex_hbm, dpart_hbm,
                 a_src_t, a_dst_t, src_t, dst_t, ex_t, m_t, z_t, denom_sh):
    c = lax.axis_index("c")
    s = lax.axis_index("s")
    wid = c * 16 + s

    pltpu.sync_copy(a_src_hbm, a_src_t)
    pltpu.sync_copy(a_dst_hbm, a_dst_t)
    pltpu.sync_copy(edge_hbm.at[0, wid], src_t)
    pltpu.sync_copy(edge_hbm.at[1, wid], dst_t)
    pltpu.sync_copy(m_hbm, m_t)

    def zinit(i, _):
        z_t[pl.ds(i * 16, 16)] = jnp.zeros((16,), jnp.float32)
        return 0
    lax.fori_loop(0, SEG // 16, zinit, 0)
    pltpu.sync_copy(z_t, denom_sh.at[pl.ds(s * SEG, SEG)])
    plsc.subcore_barrier()

    m_v = m_t[...]

    def chunk(j, _):
        for k in range(CH // 16):
            sl = pl.ds(k * 16, 16)
            sv = src_t[j, sl]
            dv = dst_t[j, sl]
            av = plsc.load_gather(a_src_t, [sv])
            bv = plsc.load_gather(a_dst_t, [dv])
            e = av + bv
            e = jnp.where(e >= 0.0, e, e * 0.2)
            ex_t[j, sl] = jnp.exp(e - m_v)
        pltpu.sync_copy(ex_t.at[j], denom_sh.at[dst_t.at[j]], add=True)
        return 0
    lax.fori_loop(0, NCH, chunk, 0)

    pltpu.sync_copy(ex_t, ex_hbm.at[wid])
    plsc.subcore_barrier()
    pltpu.sync_copy(denom_sh.at[pl.ds(s * SEG, SEG)],
                    dpart_hbm.at[c, pl.ds(s * SEG, SEG)])


DH = D // 2
NCH3 = (E // CH) // 16


@functools.partial(
    pl.kernel,
    mesh=_mesh,
    out_type=jax.ShapeDtypeStruct((2, NPAD, DH), jnp.float32),
    scratch_types=[
        pltpu.VMEM((NCH3, CH), jnp.int32),
        pltpu.VMEM((NCH3, CH), jnp.int32),
        pltpu.VMEM((NCH3, CH), jnp.float32),
        pltpu.VMEM((CH, DH), jnp.float32),
        pltpu.VMEM((CH, DH), jnp.float32),
        pltpu.VMEM((CH, DH), jnp.float32),
        pltpu.VMEM((CH, DH), jnp.float32),
        pltpu.VMEM_SHARED((NPAD, DH), jnp.float32),
        pltpu.SemaphoreType.DMA,
        pltpu.SemaphoreType.DMA,
        pltpu.SemaphoreType.DMA,
        pltpu.SemaphoreType.DMA,
    ],
    compiler_params=_sc_params,
)
def _edge_pass_b(h_hbm, edge_hbm, ex_hbm,
                 out_hbm,
                 src_t, dst_t, al_t, g0_t, g1_t, s0_t, s1_t, out_sh,
                 gsem0, gsem1, ssem0, ssem1):
    c = lax.axis_index("c")
    s = lax.axis_index("s")

    pltpu.sync_copy(edge_hbm.at[0, s], src_t)
    pltpu.sync_copy(edge_hbm.at[1, s], dst_t)
    pltpu.sync_copy(ex_hbm.at[s], al_t)

    def zrow(i, _):
        for q in range(DH // 16):
            g0_t[i, pl.ds(q * 16, 16)] = jnp.zeros((16,), jnp.float32)
        return 0
    lax.fori_loop(0, CH, zrow, 0)

    def zseg(b, _):
        pltpu.sync_copy(g0_t, out_sh.at[pl.ds(s * SEG + b * CH, CH)])
        return 0
    lax.fori_loop(0, SEG // CH, zseg, 0)

    plsc.subcore_barrier()

    gbufs = (g0_t, g1_t)
    sbufs = (s0_t, s1_t)
    gsems = (gsem0, gsem1)
    ssems = (ssem0, ssem1)

    def _proc(j, gt, st):
        jv = jnp.broadcast_to(j, (16,)).astype(jnp.int32)

        def row4(i4, _2):
            for r in range(4):
                i = i4 * 4 + r
                iv = jnp.broadcast_to(i, (16,)).astype(jnp.int32)
                ab = plsc.load_gather(al_t, [jv, iv])
                for q in range(DH // 16):
                    sl = pl.ds(q * 16, 16)
                    st[i, sl] = gt[i, sl] * ab
            return 0
        lax.fori_loop(0, CH // 4, row4, 0)

    for b in range(2):
        pltpu.async_copy(h_hbm.at[c].at[src_t.at[b]], gbufs[b], gsems[b])

    for b in range(2):
        pltpu.make_async_copy(
            h_hbm.at[c].at[src_t.at[b]], gbufs[b], gsems[b]).wait()
        _proc(b, gbufs[b], sbufs[b])
        pltpu.async_copy(sbufs[b], out_sh.at[dst_t.at[b]], ssems[b],
                         add=True)
        pltpu.async_copy(h_hbm.at[c].at[src_t.at[b + 2]], gbufs[b], gsems[b])

    def main(i, _):
        for b in range(2):
            j = i * 2 + b
            pltpu.make_async_copy(
                h_hbm.at[c].at[src_t.at[j]], gbufs[b], gsems[b]).wait()
            pltpu.make_async_copy(
                sbufs[b], out_sh.at[dst_t.at[j]], ssems[b]).wait()
            _proc(j, gbufs[b], sbufs[b])
            pltpu.async_copy(sbufs[b], out_sh.at[dst_t.at[j]], ssems[b],
                             add=True)
            pltpu.async_copy(
                h_hbm.at[c].at[src_t.at[j + 2]], gbufs[b], gsems[b])
        return 0
    lax.fori_loop(1, NCH3 // 2 - 1, main, 0)

    for b in range(2):
        j = NCH3 - 2 + b
        pltpu.make_async_copy(
            h_hbm.at[c].at[src_t.at[j]], gbufs[b], gsems[b]).wait()
        pltpu.make_async_copy(
            sbufs[b], out_sh.at[dst_t.at[j]], ssems[b]).wait()
        _proc(j, gbufs[b], sbufs[b])
        pltpu.async_copy(sbufs[b], out_sh.at[dst_t.at[j]], ssems[b],
                         add=True)

    for b in range(2):
        j = NCH3 - 2 + b
        pltpu.make_async_copy(
            sbufs[b], out_sh.at[dst_t.at[j]], ssems[b]).wait()

    plsc.subcore_barrier()

    def wb(b, _):
        r0 = s * SEG + b * CH
        pltpu.sync_copy(out_sh.at[pl.ds(r0, CH)], out_hbm.at[c, pl.ds(r0, CH)])
        return 0
    lax.fori_loop(0, SEG // CH, wb, 0)


def _dense_out_body(oa_ref, ob_ref, d0_ref, d1_ref, bc_ref, wl_ref, bl_ref,
                    y_ref):
    inv = 1.0 / (d0_ref[0] + d1_ref[0] + 1e-16)
    oa = jnp.maximum(oa_ref[0] * inv + bc_ref[:, :DH], 0.0)
    ob = jnp.maximum(ob_ref[0] * inv + bc_ref[:, DH:], 0.0)
    y = jnp.dot(oa, wl_ref[:DH, :], preferred_element_type=jnp.float32)
    y = y + jnp.dot(ob, wl_ref[DH:, :], preferred_element_type=jnp.float32)
    y_ref[...] = y + bl_ref[...]


_dense_out = pl.pallas_call(
    _dense_out_body,
    grid=(N // ROWB,),
    in_specs=[
        pl.BlockSpec((1, ROWB, DH), lambda i: (0, i, 0)),
        pl.BlockSpec((1, ROWB, DH), lambda i: (1, i, 0)),
        pl.BlockSpec((1, ROWB, 1), lambda i: (0, i, 0)),
        pl.BlockSpec((1, ROWB, 1), lambda i: (1, i, 0)),
        pl.BlockSpec((1, D), lambda i: (0, 0)),
        pl.BlockSpec((D, D), lambda i: (0, 0)),
        pl.BlockSpec((1, D), lambda i: (0, 0)),
    ],
    out_specs=pl.BlockSpec((ROWB, D), lambda i: (i, 0)),
    out_shape=jax.ShapeDtypeStruct((N, D), jnp.float32),
)


def kernel(x, edge_index, W_src, W_dst, att_src, att_dst, b_conv, W_lin, b_lin):
    h_src, a_src, a_dst, m8 = _dense_in(
        x, W_src, W_dst, att_src.reshape(D, 1), att_dst.reshape(D, 1))
    edge_a = edge_index.reshape(2, NW, NCH, CH)
    edge_b = edge_index.reshape(2, 16, NCH3, CH)
    m16 = jnp.broadcast_to(m8.reshape(-1)[:1], (16,))
    ex, dpart = _edge_pass_a(
        a_src.reshape(N), a_dst.reshape(N), edge_a, m16)
    outp = _edge_pass_b(h_src, edge_b, ex.reshape(16, NCH3, CH))
    dp = dpart.reshape(2, NPAD, 1)
    y = _dense_out(outp, outp, dp, dp, b_conv.reshape(1, D), W_lin,
                   b_lin.reshape(1, D))
    return y


# speedup vs baseline: 1.5256x; 1.5256x over previous
import functools

import jax
import jax.numpy as jnp
from jax import lax
from jax.experimental import pallas as pl
from jax.experimental.pallas import tpu as pltpu
from jax.experimental.pallas import tpu_sc as plsc

N = 10000
E = 320000
D = 128

NW = 32
EPW = E // NW
CH = 80
NCH = EPW // CH
NPAD = 10240
SEG = NPAD // 16

ROWB = 1000

_mesh = plsc.VectorSubcoreMesh(core_axis_name="c", subcore_axis_name="s")
_sc_params = pltpu.CompilerParams(needs_layout_passes=False,
                                  use_tc_tiling_on_sc=False)


def _dense_in_body(x_ref, ws_ref, wd_ref, asv_ref, adv_ref,
                   h_ref, as_ref, ad_ref, m_ref, acc):
    i = pl.program_id(0)
    h = jnp.dot(x_ref[...], ws_ref[...], preferred_element_type=jnp.float32)
    h_ref[0] = h[:, :64]
    h_ref[1] = h[:, 64:]
    a_s = jnp.dot(h, asv_ref[...], preferred_element_type=jnp.float32)
    as_ref[...] = a_s
    v_d = jnp.dot(wd_ref[...], adv_ref[...], preferred_element_type=jnp.float32)
    a_d = jnp.dot(x_ref[...], v_d, preferred_element_type=jnp.float32)
    ad_ref[...] = a_d
    bs = jnp.max(a_s)
    bd = jnp.max(a_d)

    @pl.when(i == 0)
    def _():
        acc[0] = bs
        acc[1] = bd

    @pl.when(i > 0)
    def _():
        acc[0] = jnp.maximum(acc[0], bs)
        acc[1] = jnp.maximum(acc[1], bd)

    @pl.when(i == pl.num_programs(0) - 1)
    def _():
        m = acc[0] + acc[1]
        m = jnp.where(m >= 0.0, m, m * 0.2)
        m_ref[...] = jnp.full((8, 128), m, jnp.float32)


_dense_in = pl.pallas_call(
    _dense_in_body,
    grid=(N // ROWB,),
    in_specs=[
        pl.BlockSpec((ROWB, D), lambda i: (i, 0)),
        pl.BlockSpec((D, D), lambda i: (0, 0)),
        pl.BlockSpec((D, D), lambda i: (0, 0)),
        pl.BlockSpec((D, 1), lambda i: (0, 0)),
        pl.BlockSpec((D, 1), lambda i: (0, 0)),
    ],
    out_specs=[
        pl.BlockSpec((2, ROWB, D // 2), lambda i: (0, i, 0)),
        pl.BlockSpec((ROWB, 1), lambda i: (i, 0)),
        pl.BlockSpec((ROWB, 1), lambda i: (i, 0)),
        pl.BlockSpec((8, 128), lambda i: (0, 0)),
    ],
    out_shape=[
        jax.ShapeDtypeStruct((2, N, D // 2), jnp.float32),
        jax.ShapeDtypeStruct((N, 1), jnp.float32),
        jax.ShapeDtypeStruct((N, 1), jnp.float32),
        jax.ShapeDtypeStruct((8, 128), jnp.float32),
    ],
    scratch_shapes=[pltpu.SMEM((2,), jnp.float32)],
)


@functools.partial(
    pl.kernel,
    mesh=_mesh,
    out_type=[
        jax.ShapeDtypeStruct((NW, NCH, CH), jnp.float32),
        jax.ShapeDtypeStruct((2, NPAD), jnp.float32),
    ],
    scratch_types=[
        pltpu.VMEM((N,), jnp.float32),
        pltpu.VMEM((N,), jnp.float32),
        pltpu.VMEM((NCH, CH), jnp.int32),
        pltpu.VMEM((NCH, CH), jnp.int32),
        pltpu.VMEM((NCH, CH), jnp.float32),
        pltpu.VMEM((16,), jnp.float32),
        pltpu.VMEM((SEG,), jnp.float32),
        pltpu.VMEM_SHARED((NPAD,), jnp.float32),
    ],
    compiler_params=_sc_params,
)
def _edge_pass_a(a_src_hbm, a_dst_hbm, edge_hbm, m_hbm,
                 ex_hbm, dpart_hbm,
                 a_src_t, a_dst_t, src_t, dst_t, ex_t, m_t, z_t, denom_sh):
    c = lax.axis_index("c")
    s = lax.axis_index("s")
    wid = c * 16 + s

    pltpu.sync_copy(a_src_hbm, a_src_t)
    pltpu.sync_copy(a_dst_hbm, a_dst_t)
    pltpu.sync_copy(edge_hbm.at[0, wid], src_t)
    pltpu.sync_copy(edge_hbm.at[1, wid], dst_t)
    pltpu.sync_copy(m_hbm, m_t)

    def zinit(i, _):
        z_t[pl.ds(i * 16, 16)] = jnp.zeros((16,), jnp.float32)
        return 0
    lax.fori_loop(0, SEG // 16, zinit, 0)
    pltpu.sync_copy(z_t, denom_sh.at[pl.ds(s * SEG, SEG)])
    plsc.subcore_barrier()

    m_v = m_t[...]

    def chunk(j, _):
        for k in range(CH // 16):
            sl = pl.ds(k * 16, 16)
            sv = src_t[j, sl]
            dv = dst_t[j, sl]
            av = plsc.load_gather(a_src_t, [sv])
            bv = plsc.load_gather(a_dst_t, [dv])
            e = av + bv
            e = jnp.where(e >= 0.0, e, e * 0.2)
            ex_t[j, sl] = jnp.exp(e - m_v)
        pltpu.sync_copy(ex_t.at[j], denom_sh.at[dst_t.at[j]], add=True)
        return 0
    lax.fori_loop(0, NCH, chunk, 0)

    pltpu.sync_copy(ex_t, ex_hbm.at[wid])
    plsc.subcore_barrier()
    pltpu.sync_copy(denom_sh.at[pl.ds(s * SEG, SEG)],
                    dpart_hbm.at[c, pl.ds(s * SEG, SEG)])


DH = D // 2
NCH3 = (E // CH) // 16


@functools.partial(
    pl.kernel,
    mesh=_mesh,
    out_type=jax.ShapeDtypeStruct((2, NPAD, DH), jnp.float32),
    scratch_types=[
        pltpu.VMEM((NCH3, CH), jnp.int32),
        pltpu.VMEM((NCH3, CH), jnp.int32),
        pltpu.VMEM((NCH3, CH), jnp.float32),
        pltpu.VMEM((CH, DH), jnp.float32),
        pltpu.VMEM((CH, DH), jnp.float32),
        pltpu.VMEM_SHARED((NPAD, DH), jnp.float32),
        pltpu.SemaphoreType.DMA,
        pltpu.SemaphoreType.DMA,
    ],
    compiler_params=_sc_params,
)
def _edge_pass_b(h_hbm, edge_hbm, ex_hbm,
                 out_hbm,
                 src_t, dst_t, al_t, g0_t, g1_t, out_sh,
                 gsem0, gsem1):
    c = lax.axis_index("c")
    s = lax.axis_index("s")

    pltpu.sync_copy(edge_hbm.at[0, s], src_t)
    pltpu.sync_copy(edge_hbm.at[1, s], dst_t)
    pltpu.sync_copy(ex_hbm.at[s], al_t)

    def zrow(i, _):
        for q in range(DH // 16):
            g0_t[i, pl.ds(q * 16, 16)] = jnp.zeros((16,), jnp.float32)
        return 0
    lax.fori_loop(0, CH, zrow, 0)

    def zseg(b, _):
        pltpu.sync_copy(g0_t, out_sh.at[pl.ds(s * SEG + b * CH, CH)])
        return 0
    lax.fori_loop(0, SEG // CH, zseg, 0)

    plsc.subcore_barrier()

    gbufs = (g0_t, g1_t)
    gsems = (gsem0, gsem1)

    def _proc(j, gt):
        jv = jnp.broadcast_to(j, (16,)).astype(jnp.int32)

        def row4(i4, _2):
            for r in range(4):
                i = i4 * 4 + r
                iv = jnp.broadcast_to(i, (16,)).astype(jnp.int32)
                ab = plsc.load_gather(al_t, [jv, iv])
                for q in range(DH // 16):
                    sl = pl.ds(q * 16, 16)
                    gt[i, sl] = gt[i, sl] * ab
            return 0
        lax.fori_loop(0, CH // 4, row4, 0)
        pltpu.sync_copy(gt, out_sh.at[dst_t.at[j]], add=True)

    for b in range(2):
        pltpu.async_copy(h_hbm.at[c].at[src_t.at[b]], gbufs[b], gsems[b])

    def main(i, _):
        for b in range(2):
            j = i * 2 + b
            pltpu.make_async_copy(
                h_hbm.at[c].at[src_t.at[j]], gbufs[b], gsems[b]).wait()
            _proc(j, gbufs[b])
            pltpu.async_copy(
                h_hbm.at[c].at[src_t.at[j + 2]], gbufs[b], gsems[b])
        return 0
    lax.fori_loop(0, NCH3 // 2 - 1, main, 0)

    for b in range(2):
        j = NCH3 - 2 + b
        pltpu.make_async_copy(
            h_hbm.at[c].at[src_t.at[j]], gbufs[b], gsems[b]).wait()
        _proc(j, gbufs[b])

    plsc.subcore_barrier()

    def wb(b, _):
        r0 = s * SEG + b * CH
        pltpu.sync_copy(out_sh.at[pl.ds(r0, CH)], out_hbm.at[c, pl.ds(r0, CH)])
        return 0
    lax.fori_loop(0, SEG // CH, wb, 0)


def _dense_out_body(oa_ref, ob_ref, d0_ref, d1_ref, bc_ref, wl_ref, bl_ref,
                    y_ref):
    inv = 1.0 / (d0_ref[0] + d1_ref[0] + 1e-16)
    oa = jnp.maximum(oa_ref[0] * inv + bc_ref[:, :DH], 0.0)
    ob = jnp.maximum(ob_ref[0] * inv + bc_ref[:, DH:], 0.0)
    y = jnp.dot(oa, wl_ref[:DH, :], preferred_element_type=jnp.float32)
    y = y + jnp.dot(ob, wl_ref[DH:, :], preferred_element_type=jnp.float32)
    y_ref[...] = y + bl_ref[...]


_dense_out = pl.pallas_call(
    _dense_out_body,
    grid=(N // ROWB,),
    in_specs=[
        pl.BlockSpec((1, ROWB, DH), lambda i: (0, i, 0)),
        pl.BlockSpec((1, ROWB, DH), lambda i: (1, i, 0)),
        pl.BlockSpec((1, ROWB, 1), lambda i: (0, i, 0)),
        pl.BlockSpec((1, ROWB, 1), lambda i: (1, i, 0)),
        pl.BlockSpec((1, D), lambda i: (0, 0)),
        pl.BlockSpec((D, D), lambda i: (0, 0)),
        pl.BlockSpec((1, D), lambda i: (0, 0)),
    ],
    out_specs=pl.BlockSpec((ROWB, D), lambda i: (i, 0)),
    out_shape=jax.ShapeDtypeStruct((N, D), jnp.float32),
)


def kernel(x, edge_index, W_src, W_dst, att_src, att_dst, b_conv, W_lin, b_lin):
    h_src, a_src, a_dst, m8 = _dense_in(
        x, W_src, W_dst, att_src.reshape(D, 1), att_dst.reshape(D, 1))
    edge_a = edge_index.reshape(2, NW, NCH, CH)
    edge_b = edge_index.reshape(2, 16, NCH3, CH)
    m16 = jnp.broadcast_to(m8.reshape(-1)[:1], (16,))
    ex, dpart = _edge_pass_a(
        a_src.reshape(N), a_dst.reshape(N), edge_a, m16)
    outp = _edge_pass_b(h_src, edge_b, ex.reshape(16, NCH3, CH))
    dp = dpart.reshape(2, NPAD, 1)
    y = _dense_out(outp, outp, dp, dp, b_conv.reshape(1, D), W_lin,
                   b_lin.reshape(1, D))
    return y


# speedup vs baseline: 1.7801x; 1.1668x over previous
import functools

import jax
import jax.numpy as jnp
from jax import lax
from jax.experimental import pallas as pl
from jax.experimental.pallas import tpu as pltpu
from jax.experimental.pallas import tpu_sc as plsc

N = 10000
E = 320000
D = 128

NW = 32
EPW = E // NW
CH = 80
NCH = EPW // CH
NPAD = 10240
SEG = NPAD // 16

ROWB = 1000

_mesh = plsc.VectorSubcoreMesh(core_axis_name="c", subcore_axis_name="s")
_sc_params = pltpu.CompilerParams(needs_layout_passes=False,
                                  use_tc_tiling_on_sc=False)


def _dense_in_body(x_ref, ws_ref, wd_ref, asv_ref, adv_ref,
                   h_ref, as_ref, ad_ref, m_ref, ms_ref, acc):
    i = pl.program_id(0)
    h = jnp.dot(x_ref[...], ws_ref[...], preferred_element_type=jnp.float32)
    h_ref[0] = h[:, :64]
    h_ref[1] = h[:, 64:]
    a_s = jnp.dot(h, asv_ref[...], preferred_element_type=jnp.float32)
    as_ref[...] = a_s
    v_d = jnp.dot(wd_ref[...], adv_ref[...], preferred_element_type=jnp.float32)
    a_d = jnp.dot(x_ref[...], v_d, preferred_element_type=jnp.float32)
    ad_ref[...] = a_d
    bs = jnp.max(a_s)
    bd = jnp.max(a_d)

    @pl.when(i == 0)
    def _():
        acc[0] = bs
        acc[1] = bd

    @pl.when(i > 0)
    def _():
        acc[0] = jnp.maximum(acc[0], bs)
        acc[1] = jnp.maximum(acc[1], bd)

    @pl.when(i == pl.num_programs(0) - 1)
    def _():
        m = acc[0] + acc[1]
        m = jnp.where(m >= 0.0, m, m * 0.2)
        m_ref[...] = jnp.full((8, 128), m, jnp.float32)
        ms_ref[...] = jnp.full((8, 128), acc[0], jnp.float32)


_dense_in = pl.pallas_call(
    _dense_in_body,
    grid=(N // ROWB,),
    in_specs=[
        pl.BlockSpec((ROWB, D), lambda i: (i, 0)),
        pl.BlockSpec((D, D), lambda i: (0, 0)),
        pl.BlockSpec((D, D), lambda i: (0, 0)),
        pl.BlockSpec((D, 1), lambda i: (0, 0)),
        pl.BlockSpec((D, 1), lambda i: (0, 0)),
    ],
    out_specs=[
        pl.BlockSpec((2, ROWB, D // 2), lambda i: (0, i, 0)),
        pl.BlockSpec((ROWB, 1), lambda i: (i, 0)),
        pl.BlockSpec((ROWB, 1), lambda i: (i, 0)),
        pl.BlockSpec((8, 128), lambda i: (0, 0)),
        pl.BlockSpec((8, 128), lambda i: (0, 0)),
    ],
    out_shape=[
        jax.ShapeDtypeStruct((2, N, D // 2), jnp.float32),
        jax.ShapeDtypeStruct((N, 1), jnp.float32),
        jax.ShapeDtypeStruct((N, 1), jnp.float32),
        jax.ShapeDtypeStruct((8, 128), jnp.float32),
        jax.ShapeDtypeStruct((8, 128), jnp.float32),
    ],
    scratch_shapes=[pltpu.SMEM((2,), jnp.float32)],
)


DH = D // 2


def _scale_body(h_ref, as_ref, ms_ref, g_ref):
    ms = ms_ref[0, 0]
    a = as_ref[...]
    f_p = jnp.exp(a - ms)
    f_n = jnp.exp(0.2 * (a - ms))
    for c in range(2):
        g_ref[c, 0] = h_ref[c] * f_p
        g_ref[c, 1] = h_ref[c] * f_n


_scale = pl.pallas_call(
    _scale_body,
    grid=(N // ROWB,),
    in_specs=[
        pl.BlockSpec((2, ROWB, DH), lambda i: (0, i, 0)),
        pl.BlockSpec((ROWB, 1), lambda i: (i, 0)),
        pl.BlockSpec((8, 128), lambda i: (0, 0)),
    ],
    out_specs=pl.BlockSpec((2, 2, ROWB, DH), lambda i: (0, 0, i, 0)),
    out_shape=jax.ShapeDtypeStruct((2, 2, N, DH), jnp.float32),
)


@functools.partial(
    pl.kernel,
    mesh=_mesh,
    out_type=[
        jax.ShapeDtypeStruct((2, NPAD), jnp.float32),
        jax.ShapeDtypeStruct((NW, NCH, CH), jnp.int32),
        jax.ShapeDtypeStruct((NW, NCH, CH), jnp.int32),
    ],
    scratch_types=[
        pltpu.VMEM((N,), jnp.float32),
        pltpu.VMEM((N,), jnp.float32),
        pltpu.VMEM((NCH, CH), jnp.int32),
        pltpu.VMEM((NCH, CH), jnp.int32),
        pltpu.VMEM((NCH, CH), jnp.float32),
        pltpu.VMEM((NCH, CH), jnp.int32),
        pltpu.VMEM((NCH, CH), jnp.int32),
        pltpu.VMEM((16,), jnp.float32),
        pltpu.VMEM((SEG,), jnp.float32),
        pltpu.VMEM_SHARED((NPAD,), jnp.float32),
    ],
    compiler_params=_sc_params,
)
def _edge_pass_a(a_src_hbm, a_dst_hbm, edge_hbm, m_hbm,
                 dpart_hbm, src2_hbm, dst2_hbm,
                 a_src_t, a_dst_t, src_t, dst_t, ex_t, src2_t, dst2_t,
                 m_t, z_t, denom_sh):
    c = lax.axis_index("c")
    s = lax.axis_index("s")
    wid = c * 16 + s

    pltpu.sync_copy(a_src_hbm, a_src_t)
    pltpu.sync_copy(a_dst_hbm, a_dst_t)
    pltpu.sync_copy(edge_hbm.at[0, wid], src_t)
    pltpu.sync_copy(edge_hbm.at[1, wid], dst_t)
    pltpu.sync_copy(m_hbm, m_t)

    def zinit(i, _):
        z_t[pl.ds(i * 16, 16)] = jnp.zeros((16,), jnp.float32)
        return 0
    lax.fori_loop(0, SEG // 16, zinit, 0)
    pltpu.sync_copy(z_t, denom_sh.at[pl.ds(s * SEG, SEG)])
    plsc.subcore_barrier()

    m_v = m_t[...]

    def chunk(j, _):
        for k in range(CH // 16):
            sl = pl.ds(k * 16, 16)
            sv = src_t[j, sl]
            dv = dst_t[j, sl]
            av = plsc.load_gather(a_src_t, [sv])
            bv = plsc.load_gather(a_dst_t, [dv])
            e = av + bv
            neg = e < 0.0
            src2_t[j, sl] = jnp.where(neg, sv + N, sv)
            dst2_t[j, sl] = jnp.where(neg, dv + N, dv)
            el = jnp.where(neg, e * 0.2, e)
            ex_t[j, sl] = jnp.exp(el - m_v)
        pltpu.sync_copy(ex_t.at[j], denom_sh.at[dst_t.at[j]], add=True)
        return 0
    lax.fori_loop(0, NCH, chunk, 0)

    pltpu.sync_copy(src2_t, src2_hbm.at[wid])
    pltpu.sync_copy(dst2_t, dst2_hbm.at[wid])
    plsc.subcore_barrier()
    pltpu.sync_copy(denom_sh.at[pl.ds(s * SEG, SEG)],
                    dpart_hbm.at[c, pl.ds(s * SEG, SEG)])


NCH3 = (E // CH) // 16
NACC = 2 * N
WSEG = NACC // 16
WB = 50


@functools.partial(
    pl.kernel,
    mesh=_mesh,
    out_type=jax.ShapeDtypeStruct((2, NACC, DH), jnp.float32),
    scratch_types=[
        pltpu.VMEM((NCH3, CH), jnp.int32),
        pltpu.VMEM((2, CH), jnp.int32),
        pltpu.VMEM((CH, DH), jnp.float32),
        pltpu.VMEM((CH, DH), jnp.float32),
        pltpu.VMEM_SHARED((NACC, DH), jnp.float32),
        pltpu.SemaphoreType.DMA,
        pltpu.SemaphoreType.DMA,
        pltpu.SemaphoreType.DMA,
        pltpu.SemaphoreType.DMA,
    ],
    compiler_params=_sc_params,
)
def _edge_pass_b(g_hbm, src2_hbm, dst2_hbm,
                 out_hbm,
                 src_t, d2_t, g0_t, g1_t, out_sh,
                 gsem0, gsem1, dsem0, dsem1):
    c = lax.axis_index("c")
    s = lax.axis_index("s")

    pltpu.sync_copy(src2_hbm.at[s], src_t)

    def zrow(i, _):
        for q in range(DH // 16):
            g0_t[i, pl.ds(q * 16, 16)] = jnp.zeros((16,), jnp.float32)
        return 0
    lax.fori_loop(0, CH, zrow, 0)

    def zseg(b, _):
        pltpu.sync_copy(g0_t.at[pl.ds(0, WB)],
                        out_sh.at[pl.ds(s * WSEG + b * WB, WB)])
        return 0
    lax.fori_loop(0, WSEG // WB, zseg, 0)

    plsc.subcore_barrier()

    gbufs = (g0_t, g1_t)
    gsems = (gsem0, gsem1)
    dsems = (dsem0, dsem1)

    for b in range(2):
        pltpu.async_copy(g_hbm.at[c].at[src_t.at[b]], gbufs[b], gsems[b])
        pltpu.async_copy(dst2_hbm.at[s, b], d2_t.at[b], dsems[b])

    def main(i, _):
        for b in range(2):
            j = i * 2 + b
            pltpu.make_async_copy(
                g_hbm.at[c].at[src_t.at[j]], gbufs[b], gsems[b]).wait()
            pltpu.make_async_copy(
                dst2_hbm.at[s, j], d2_t.at[b], dsems[b]).wait()
            pltpu.sync_copy(gbufs[b], out_sh.at[d2_t.at[b]], add=True)
            pltpu.async_copy(
                g_hbm.at[c].at[src_t.at[j + 2]], gbufs[b], gsems[b])
            pltpu.async_copy(dst2_hbm.at[s, j + 2], d2_t.at[b], dsems[b])
        return 0
    lax.fori_loop(0, NCH3 // 2 - 1, main, 0)

    for b in range(2):
        j = NCH3 - 2 + b
        pltpu.make_async_copy(
            g_hbm.at[c].at[src_t.at[j]], gbufs[b], gsems[b]).wait()
        pltpu.make_async_copy(
            dst2_hbm.at[s, j], d2_t.at[b], dsems[b]).wait()
        pltpu.sync_copy(gbufs[b], out_sh.at[d2_t.at[b]], add=True)

    plsc.subcore_barrier()

    def wb(b, _):
        r0 = s * WSEG + b * WB
        pltpu.sync_copy(out_sh.at[pl.ds(r0, WB)], out_hbm.at[c, pl.ds(r0, WB)])
        return 0
    lax.fori_loop(0, WSEG // WB, wb, 0)


def _dense_out_body(pa_ref, na_ref, pb_ref, nb_ref, ad_ref, m_ref, ms_ref,
                    d0_ref, d1_ref, bc_ref, wl_ref, bl_ref, y_ref):
    m = m_ref[0, 0]
    ms = ms_ref[0, 0]
    a = ad_ref[...]
    f_p = jnp.exp(a - (m - ms))
    f_n = jnp.exp(0.2 * a - (m - 0.2 * ms))
    inv = 1.0 / (d0_ref[0] + d1_ref[0] + 1e-16)
    oa = jnp.maximum((pa_ref[0] * f_p + na_ref[0] * f_n) * inv
                     + bc_ref[:, :DH], 0.0)
    ob = jnp.maximum((pb_ref[0] * f_p + nb_ref[0] * f_n) * inv
                     + bc_ref[:, DH:], 0.0)
    y = jnp.dot(oa, wl_ref[:DH, :], preferred_element_type=jnp.float32)
    y = y + jnp.dot(ob, wl_ref[DH:, :], preferred_element_type=jnp.float32)
    y_ref[...] = y + bl_ref[...]


_dense_out = pl.pallas_call(
    _dense_out_body,
    grid=(N // ROWB,),
    in_specs=[
        pl.BlockSpec((1, ROWB, DH), lambda i: (0, i, 0)),
        pl.BlockSpec((1, ROWB, DH), lambda i: (0, N // ROWB + i, 0)),
        pl.BlockSpec((1, ROWB, DH), lambda i: (1, i, 0)),
        pl.BlockSpec((1, ROWB, DH), lambda i: (1, N // ROWB + i, 0)),
        pl.BlockSpec((ROWB, 1), lambda i: (i, 0)),
        pl.BlockSpec((8, 128), lambda i: (0, 0)),
        pl.BlockSpec((8, 128), lambda i: (0, 0)),
        pl.BlockSpec((1, ROWB, 1), lambda i: (0, i, 0)),
        pl.BlockSpec((1, ROWB, 1), lambda i: (1, i, 0)),
        pl.BlockSpec((1, D), lambda i: (0, 0)),
        pl.BlockSpec((D, D), lambda i: (0, 0)),
        pl.BlockSpec((1, D), lambda i: (0, 0)),
    ],
    out_specs=pl.BlockSpec((ROWB, D), lambda i: (i, 0)),
    out_shape=jax.ShapeDtypeStruct((N, D), jnp.float32),
)


def kernel(x, edge_index, W_src, W_dst, att_src, att_dst, b_conv, W_lin, b_lin):
    h_src, a_src, a_dst, m8, ms8 = _dense_in(
        x, W_src, W_dst, att_src.reshape(D, 1), att_dst.reshape(D, 1))
    g = _scale(h_src, a_src, ms8)
    edge_a = edge_index.reshape(2, NW, NCH, CH)
    m16 = jnp.broadcast_to(m8.reshape(-1)[:1], (16,))
    dpart, src2, dst2 = _edge_pass_a(
        a_src.reshape(N), a_dst.reshape(N), edge_a, m16)
    acc = _edge_pass_b(g.reshape(2, 2 * N, DH),
                       src2.reshape(16, NCH3, CH),
                       dst2.reshape(16, NCH3, CH))
    dp = dpart.reshape(2, NPAD, 1)
    y = _dense_out(acc, acc, acc, acc, a_dst, m8, ms8, dp, dp,
                   b_conv.reshape(1, D), W_lin, b_lin.reshape(1, D))
    return y
